# baseline (speedup 1.0000x reference)
import jax
import jax.numpy as jnp
from jax.experimental import pallas as pl
from jax.experimental.pallas import tpu as pltpu

_K = 7
_KP = 8
_D = 512
_B = 16384
_R0 = 2048
_NB0 = _B // _R0
_R1 = 4096
_NB1 = _B // _R1
_MIX = 0.3
_KEEP = 1.0 - _MIX
_NEG = -1.0e30


def _fused_kernel(dom_ref, x_ref, out_ref, xbuf, cent_s, cent3_s, b2m_s,
                  cnt_s):
    s = pl.program_id(0)

    @pl.when(s < _NB0)
    def _phase0():
        i = s
        x = x_ref[...]
        xbuf[pl.ds(i * _R0, _R0), :] = x
        dom = dom_ref[0]
        k8 = jax.lax.broadcasted_iota(jnp.int32, (_KP, 1), 0)
        oh = (dom == k8).astype(jnp.float32)
        psum = jax.lax.dot_general(
            oh, x, (((1,), (0,)), ((), ())),
            preferred_element_type=jnp.float32)
        pcnt = jnp.sum(oh, axis=1, keepdims=True)

        @pl.when(i == 0)
        def _():
            cent_s[...] = psum
            cnt_s[...] = pcnt

        @pl.when(i > 0)
        def _():
            cent_s[...] += psum
            cnt_s[...] += pcnt

        @pl.when(i == _NB0 - 1)
        def _():
            cnt = cnt_s[...]
            cent = jnp.where(
                cnt > 0.0, cent_s[...] / jnp.maximum(cnt, 1.0), 0.0)
            cent_s[...] = cent
            cent3_s[...] = _MIX * cent
            b2 = jnp.sum(cent * cent, axis=1, keepdims=True)
            b2m_s[...] = jnp.where(cnt > 0.0, b2, _NEG)

    @pl.when(s >= _NB0)
    def _phase1():
        j = s - _NB0
        x = xbuf[pl.ds(j * _R1, _R1), :]
        cent = cent_s[...]
        xcT = jax.lax.dot_general(
            cent, x, (((1,), (1,)), ((), ())),
            preferred_element_type=jnp.float32)
        score = b2m_s[...] - 2.0 * xcT
        m4 = jnp.maximum(score[0:4], score[4:8])
        m2 = jnp.maximum(m4[0:2], m4[2:4])
        m1 = jnp.maximum(m2[0:1], m2[1:2])
        k8 = jax.lax.broadcasted_iota(jnp.int32, (_KP, 1), 0)
        kk = jnp.where(score == m1, k8, _KP)
        i4 = jnp.minimum(kk[0:4], kk[4:8])
        i2 = jnp.minimum(i4[0:2], i4[2:4])
        idx = jnp.minimum(i2[0:1], i2[1:2])
        ohsel = (k8 == idx).astype(jnp.float32)
        hardest3 = jax.lax.dot_general(
            ohsel, cent3_s[...], (((0,), (0,)), ((), ())),
            preferred_element_type=jnp.float32)
        out_ref[...] = _KEEP * x + hardest3


@jax.jit
def kernel(c_vt, domains):
    dom3 = domains.reshape(_NB0, 1, _R0)
    out = pl.pallas_call(
        _fused_kernel,
        grid=(_NB0 + _NB1,),
        in_specs=[
            pl.BlockSpec(
                (1, 1, _R0),
                lambda s: (jnp.minimum(s, _NB0 - 1), 0, 0)),
            pl.BlockSpec(
                (_R0, _D),
                lambda s: (jnp.minimum(s, _NB0 - 1), 0)),
        ],
        out_specs=pl.BlockSpec(
            (_R1, _D), lambda s: (jnp.maximum(s - _NB0, 0), 0)),
        out_shape=jax.ShapeDtypeStruct((_B, _D), jnp.float32),
        scratch_shapes=[
            pltpu.VMEM((_B, _D), jnp.float32),
            pltpu.VMEM((_KP, _D), jnp.float32),
            pltpu.VMEM((_KP, _D), jnp.float32),
            pltpu.VMEM((_KP, 1), jnp.float32),
            pltpu.VMEM((_KP, 1), jnp.float32),
        ],
    )(dom3, c_vt)
    return out


# speedup vs baseline: 1.0280x; 1.0280x over previous
import jax
import jax.numpy as jnp
from jax.experimental import pallas as pl
from jax.experimental.pallas import tpu as pltpu

_K = 7
_KP = 8
_D = 512
_B = 16384
_R0 = 4096
_NB0 = _B // _R0
_R1 = 2048
_NB1 = _B // _R1
_MIX = 0.3
_KEEP = 1.0 - _MIX
_NEG = -1.0e30


def _fused_kernel(dom_ref, x_ref, out_ref, xbuf, cent_s, cent3_s, b2m_s,
                  cnt_s):
    s = pl.program_id(0)

    @pl.when(s < _NB0)
    def _phase0():
        i = s
        x = x_ref[...]
        xbuf[pl.ds(i * _R0, _R0), :] = x
        dom = dom_ref[0]
        k8 = jax.lax.broadcasted_iota(jnp.int32, (_KP, 1), 0)
        oh = (dom == k8).astype(jnp.float32)
        psum = jax.lax.dot_general(
            oh, x, (((1,), (0,)), ((), ())),
            preferred_element_type=jnp.float32)
        pcnt = jnp.sum(oh, axis=1, keepdims=True)

        @pl.when(i == 0)
        def _():
            cent_s[...] = psum
            cnt_s[...] = pcnt

        @pl.when(i > 0)
        def _():
            cent_s[...] += psum
            cnt_s[...] += pcnt

        @pl.when(i == _NB0 - 1)
        def _():
            cnt = cnt_s[...]
            cent = jnp.where(
                cnt > 0.0, cent_s[...] / jnp.maximum(cnt, 1.0), 0.0)
            cent_s[...] = cent
            cent3_s[...] = _MIX * cent
            b2 = jnp.sum(cent * cent, axis=1, keepdims=True)
            b2m_s[...] = jnp.where(cnt > 0.0, b2, _NEG)

    @pl.when(s >= _NB0)
    def _phase1():
        j = s - _NB0
        x = xbuf[pl.ds(j * _R1, _R1), :]
        cent = cent_s[...]
        xcT = jax.lax.dot_general(
            cent, x, (((1,), (1,)), ((), ())),
            preferred_element_type=jnp.float32)
        score = b2m_s[...] - 2.0 * xcT
        m4 = jnp.maximum(score[0:4], score[4:8])
        m2 = jnp.maximum(m4[0:2], m4[2:4])
        m1 = jnp.maximum(m2[0:1], m2[1:2])
        k8 = jax.lax.broadcasted_iota(jnp.int32, (_KP, 1), 0)
        kk = jnp.where(score == m1, k8, _KP)
        i4 = jnp.minimum(kk[0:4], kk[4:8])
        i2 = jnp.minimum(i4[0:2], i4[2:4])
        idx = jnp.minimum(i2[0:1], i2[1:2])
        ohsel = (k8 == idx).astype(jnp.float32)
        hardest3 = jax.lax.dot_general(
            ohsel, cent3_s[...], (((0,), (0,)), ((), ())),
            preferred_element_type=jnp.float32)
        out_ref[...] = _KEEP * x + hardest3


@jax.jit
def kernel(c_vt, domains):
    dom3 = domains.reshape(_NB0, 1, _R0)
    out = pl.pallas_call(
        _fused_kernel,
        grid=(_NB0 + _NB1,),
        in_specs=[
            pl.BlockSpec(
                (1, 1, _R0),
                lambda s: (jnp.minimum(s, _NB0 - 1), 0, 0)),
            pl.BlockSpec(
                (_R0, _D),
                lambda s: (jnp.minimum(s, _NB0 - 1), 0)),
        ],
        out_specs=pl.BlockSpec(
            (_R1, _D), lambda s: (jnp.maximum(s - _NB0, 0), 0)),
        out_shape=jax.ShapeDtypeStruct((_B, _D), jnp.float32),
        scratch_shapes=[
            pltpu.VMEM((_B, _D), jnp.float32),
            pltpu.VMEM((_KP, _D), jnp.float32),
            pltpu.VMEM((_KP, _D), jnp.float32),
            pltpu.VMEM((_KP, 1), jnp.float32),
            pltpu.VMEM((_KP, 1), jnp.float32),
        ],
    )(dom3, c_vt)
    return out
